# Initial kernel scaffold; baseline (speedup 1.0000x reference)
import jax
import jax.numpy as jnp
from jax.experimental import pallas as pl


def kernel(x, edge_index, edge_attr, W1, b1, W2, b2):
    raise NotImplementedError("write your pallas kernel here")


# speedup vs baseline: 3.4469x; 3.4469x over previous
import functools

import jax
import jax.numpy as jnp
from jax import lax
from jax.experimental import pallas as pl
from jax.experimental.pallas import tpu as pltpu
from jax.experimental.pallas import tpu_sc as plsc

NC = 2
NS = 16
CHUNK = 128
SUB = 32
LANES = 16


def _make_sc_scatter(nch, n_pad, d, dtype):
    rows_per_tile = n_pad // NS
    stage_rows = CHUNK
    mesh = plsc.VectorSubcoreMesh(core_axis_name="c", subcore_axis_name="s")

    @functools.partial(
        pl.kernel,
        out_type=jax.ShapeDtypeStruct((NC, n_pad, d), dtype),
        mesh=mesh,
        scratch_types=[
            pltpu.VMEM((SUB, CHUNK), jnp.int32),
            pltpu.VMEM((SUB, CHUNK), jnp.int32),
            pltpu.VMEM((SUB, CHUNK), dtype),
            pltpu.VMEM((CHUNK, d), dtype),
            pltpu.VMEM_SHARED((n_pad, d), dtype),
            pltpu.SemaphoreType.DMA,
        ],
    )
    def sc_scatter(gidx_hbm, sidx_hbm, attr_hbm, x_hbm, out_hbm,
                   gidx_v, sidx_v, attr_v, rows_v, acc_sh, sem):
        c = lax.axis_index("c")
        s = lax.axis_index("s")

        def zero_row(i, carry):
            for q in range(d // LANES):
                rows_v[i, pl.ds(q * LANES, LANES)] = jnp.zeros((LANES,), dtype)
            return carry
        lax.fori_loop(0, stage_rows, zero_row, 0)
        base = s * rows_per_tile
        for k in range(rows_per_tile // stage_rows):
            pltpu.sync_copy(rows_v, acc_sh.at[pl.ds(base + k * stage_rows, stage_rows)])
        plsc.subcore_barrier()

        def block_body(b, carry):
            pltpu.sync_copy(gidx_hbm.at[c, s, pl.ds(b * SUB, SUB)], gidx_v)
            pltpu.sync_copy(sidx_hbm.at[c, s, pl.ds(b * SUB, SUB)], sidx_v)
            pltpu.sync_copy(attr_hbm.at[s, pl.ds(b * SUB, SUB)], attr_v)

            def chunk_body(j, ccarry):
                pltpu.async_copy(x_hbm.at[gidx_v.at[j]], rows_v, sem).wait()

                def edge_body(g, icarry):
                    a16 = attr_v[j, pl.ds(g * LANES, LANES)]
                    for t in range(LANES):
                        av = jnp.full((LANES,), a16[t], dtype)
                        i = g * LANES + t
                        for q in range(d // LANES):
                            sl = pl.ds(q * LANES, LANES)
                            rows_v[i, sl] = rows_v[i, sl] * av
                    return icarry
                lax.fori_loop(0, CHUNK // LANES, edge_body, 0)

                pltpu.sync_copy(rows_v, acc_sh.at[sidx_v.at[j]], add=True)
                return ccarry
            lax.fori_loop(0, SUB, chunk_body, 0)
            return carry
        lax.fori_loop(0, nch // SUB, block_body, 0)
        plsc.subcore_barrier()

        for k in range(rows_per_tile // stage_rows):
            off = base + k * stage_rows
            pltpu.sync_copy(acc_sh.at[pl.ds(off, stage_rows)], rows_v)
            pltpu.sync_copy(rows_v, out_hbm.at[c, pl.ds(off, stage_rows)])

    return sc_scatter


def _mlp_body(mi_ref, mo_ref, x_ref, w1a_ref, w1b_ref, w1c_ref, b1_ref,
              w2_ref, b2_ref, o_ref):
    acc = jnp.dot(mi_ref[...], w1a_ref[...], preferred_element_type=jnp.float32)
    acc = acc + jnp.dot(mo_ref[...], w1b_ref[...], preferred_element_type=jnp.float32)
    acc = acc + jnp.dot(x_ref[...], w1c_ref[...], preferred_element_type=jnp.float32)
    h = jnp.tanh(acc + b1_ref[...])
    o = jnp.dot(h, w2_ref[...], preferred_element_type=jnp.float32) + b2_ref[...]
    o_ref[...] = jnp.tanh(o)


def _mlp(mi, mo, x, W1, b1, W2, b2):
    n, d = x.shape
    blk = 400
    grid = n // blk
    row_spec = pl.BlockSpec((blk, d), lambda i: (i, 0))
    full = lambda shape: pl.BlockSpec(shape, lambda i: tuple(0 for _ in shape))
    return pl.pallas_call(
        _mlp_body,
        grid=(grid,),
        in_specs=[
            row_spec, row_spec, row_spec,
            full((d, d)), full((d, d)), full((d, d)), full((1, d)),
            full((d, d)), full((1, d)),
        ],
        out_specs=row_spec,
        out_shape=jax.ShapeDtypeStruct((n, d), x.dtype),
    )(mi, mo, x, W1[:d], W1[d:2 * d], W1[2 * d:], b1.reshape(1, d),
      W2, b2.reshape(1, d))


def kernel(x, edge_index, edge_attr, W1, b1, W2, b2):
    n, d = x.shape
    e = edge_index.shape[1]
    per_tile = -(-e // NS)
    nch = -(-per_tile // (CHUNK * SUB)) * SUB
    e_pad = NS * nch * CHUNK
    pad = e_pad - e

    row = edge_index[0]
    col = edge_index[1]
    attr = edge_attr[:, 0]
    if pad:
        zi = jnp.zeros((pad,), jnp.int32)
        row = jnp.concatenate([row, zi])
        col = jnp.concatenate([col, zi])
        attr = jnp.concatenate([attr, jnp.zeros((pad,), attr.dtype)])

    gidx = jnp.stack([row, col]).reshape(NC, NS, nch, CHUNK)
    sidx = jnp.stack([col, row]).reshape(NC, NS, nch, CHUNK)
    attr_r = attr.reshape(NS, nch, CHUNK)

    rpt = -(-(-(-n // NS)) // CHUNK) * CHUNK
    n_pad = NS * rpt
    mimo = _make_sc_scatter(nch, n_pad, d, x.dtype)(gidx, sidx, attr_r, x)
    return _mlp(mimo[0, :n], mimo[1, :n], x, W1, b1, W2, b2)


# speedup vs baseline: 3.8618x; 1.1204x over previous
import functools

import jax
import jax.numpy as jnp
from jax import lax
from jax.experimental import pallas as pl
from jax.experimental.pallas import tpu as pltpu
from jax.experimental.pallas import tpu_sc as plsc

NC = 2
NS = 16
CHUNK = 128
SUB = 32
LANES = 16


def _make_sc_scatter(nch, n_pad, d, dtype):
    rows_per_tile = n_pad // NS
    stage_rows = CHUNK
    mesh = plsc.VectorSubcoreMesh(core_axis_name="c", subcore_axis_name="s")

    @functools.partial(
        pl.kernel,
        out_type=jax.ShapeDtypeStruct((NC, n_pad, d), dtype),
        mesh=mesh,
        scratch_types=[
            pltpu.VMEM((SUB, CHUNK), jnp.int32),
            pltpu.VMEM((SUB, CHUNK), jnp.int32),
            pltpu.VMEM((SUB, CHUNK), dtype),
            pltpu.VMEM((CHUNK, d), dtype),
            pltpu.VMEM((CHUNK, d), dtype),
            pltpu.VMEM_SHARED((n_pad, d), dtype),
            pltpu.SemaphoreType.DMA,
            pltpu.SemaphoreType.DMA,
        ],
    )
    def sc_scatter(gidx_hbm, sidx_hbm, attr_hbm, x_hbm, out_hbm,
                   gidx_v, sidx_v, attr_v, rows_a, rows_b, acc_sh,
                   gsem_a, gsem_b):
        c = lax.axis_index("c")
        s = lax.axis_index("s")

        def zero_row(i, carry):
            for q in range(d // LANES):
                rows_a[i, pl.ds(q * LANES, LANES)] = jnp.zeros((LANES,), dtype)
            return carry
        lax.fori_loop(0, stage_rows, zero_row, 0)
        base = s * rows_per_tile
        for k in range(rows_per_tile // stage_rows):
            pltpu.sync_copy(rows_a, acc_sh.at[pl.ds(base + k * stage_rows, stage_rows)])
        plsc.subcore_barrier()

        def scale(rows_v, j):
            def edge_body(g, icarry):
                a16 = attr_v[j, pl.ds(g * LANES, LANES)]
                for t in range(LANES):
                    av = jnp.full((LANES,), a16[t], dtype)
                    i = g * LANES + t
                    for q in range(d // LANES):
                        sl = pl.ds(q * LANES, LANES)
                        rows_v[i, sl] = rows_v[i, sl] * av
                return icarry
            lax.fori_loop(0, CHUNK // LANES, edge_body, 0)

        def step(rows_v, gsem, j, prefetch_j):
            pltpu.make_async_copy(x_hbm.at[gidx_v.at[j]], rows_v, gsem).wait()
            scale(rows_v, j)
            pltpu.sync_copy(rows_v, acc_sh.at[sidx_v.at[j]], add=True)
            if prefetch_j is not None:
                pltpu.async_copy(x_hbm.at[gidx_v.at[prefetch_j]], rows_v, gsem)

        for b in range(nch // SUB):
            pltpu.sync_copy(gidx_hbm.at[c, s, pl.ds(b * SUB, SUB)], gidx_v)
            pltpu.sync_copy(sidx_hbm.at[c, s, pl.ds(b * SUB, SUB)], sidx_v)
            pltpu.sync_copy(attr_hbm.at[s, pl.ds(b * SUB, SUB)], attr_v)
            pltpu.async_copy(x_hbm.at[gidx_v.at[0]], rows_a, gsem_a)
            pltpu.async_copy(x_hbm.at[gidx_v.at[1]], rows_b, gsem_b)

            def pair(j2, carry):
                j = j2 * 2
                step(rows_a, gsem_a, j, j + 2)
                step(rows_b, gsem_b, j + 1, j + 3)
                return carry
            lax.fori_loop(0, (SUB - 2) // 2, pair, 0)
            step(rows_a, gsem_a, SUB - 2, None)
            step(rows_b, gsem_b, SUB - 1, None)
        plsc.subcore_barrier()

        for k in range(rows_per_tile // stage_rows):
            off = base + k * stage_rows
            pltpu.sync_copy(acc_sh.at[pl.ds(off, stage_rows)], rows_a)
            pltpu.sync_copy(rows_a, out_hbm.at[c, pl.ds(off, stage_rows)])

    return sc_scatter


def _mlp_body(mi_ref, mo_ref, x_ref, w1a_ref, w1b_ref, w1c_ref, b1_ref,
              w2_ref, b2_ref, o_ref):
    acc = jnp.dot(mi_ref[...], w1a_ref[...], preferred_element_type=jnp.float32)
    acc = acc + jnp.dot(mo_ref[...], w1b_ref[...], preferred_element_type=jnp.float32)
    acc = acc + jnp.dot(x_ref[...], w1c_ref[...], preferred_element_type=jnp.float32)
    h = jnp.tanh(acc + b1_ref[...])
    o = jnp.dot(h, w2_ref[...], preferred_element_type=jnp.float32) + b2_ref[...]
    o_ref[...] = jnp.tanh(o)


def _mlp(mi, mo, x, W1, b1, W2, b2):
    n, d = x.shape
    blk = 400
    grid = n // blk
    row_spec = pl.BlockSpec((blk, d), lambda i: (i, 0))
    full = lambda shape: pl.BlockSpec(shape, lambda i: tuple(0 for _ in shape))
    return pl.pallas_call(
        _mlp_body,
        grid=(grid,),
        in_specs=[
            row_spec, row_spec, row_spec,
            full((d, d)), full((d, d)), full((d, d)), full((1, d)),
            full((d, d)), full((1, d)),
        ],
        out_specs=row_spec,
        out_shape=jax.ShapeDtypeStruct((n, d), x.dtype),
    )(mi, mo, x, W1[:d], W1[d:2 * d], W1[2 * d:], b1.reshape(1, d),
      W2, b2.reshape(1, d))


def kernel(x, edge_index, edge_attr, W1, b1, W2, b2):
    n, d = x.shape
    e = edge_index.shape[1]
    per_tile = -(-e // NS)
    nch = -(-per_tile // (CHUNK * SUB)) * SUB
    e_pad = NS * nch * CHUNK
    pad = e_pad - e

    row = edge_index[0]
    col = edge_index[1]
    attr = edge_attr[:, 0]
    if pad:
        zi = jnp.zeros((pad,), jnp.int32)
        row = jnp.concatenate([row, zi])
        col = jnp.concatenate([col, zi])
        attr = jnp.concatenate([attr, jnp.zeros((pad,), attr.dtype)])

    gidx = jnp.stack([row, col]).reshape(NC, NS, nch, CHUNK)
    sidx = jnp.stack([col, row]).reshape(NC, NS, nch, CHUNK)
    attr_r = attr.reshape(NS, nch, CHUNK)

    rpt = -(-(-(-n // NS)) // CHUNK) * CHUNK
    n_pad = NS * rpt
    mimo = _make_sc_scatter(nch, n_pad, d, x.dtype)(gidx, sidx, attr_r, x)
    return _mlp(mimo[0, :n], mimo[1, :n], x, W1, b1, W2, b2)
